# baseline (speedup 1.0000x reference)
import jax
import jax.numpy as jnp
from jax.experimental import pallas as pl

_NPF = 256
_NPX = 26
_NPY = 230


def _pos_kernel(ce_ref, re_ref, out_ref):
    h = out_ref.shape[2]
    w = out_ref.shape[3]
    ce_t = ce_ref[:].T
    re_t = re_ref[:].T
    out_ref[0, 0:_NPX] = jnp.broadcast_to(ce_t[:, None, :], (_NPX, h, w))
    out_ref[0, _NPX:_NPF] = jnp.broadcast_to(re_t[:, :, None], (_NPY, h, w))


def kernel(x, row_embed, col_embed):
    b = x.shape[0]
    h, w = x.shape[-2], x.shape[-1]
    ce = col_embed[:w]
    re = row_embed[:h]
    out = pl.pallas_call(
        _pos_kernel,
        grid=(b,),
        in_specs=[
            pl.BlockSpec((w, _NPX), lambda i: (0, 0)),
            pl.BlockSpec((h, _NPY), lambda i: (0, 0)),
        ],
        out_specs=pl.BlockSpec((1, _NPF, h, w), lambda i: (i, 0, 0, 0)),
        out_shape=jax.ShapeDtypeStruct((b, _NPF, h, w), jnp.float32),
    )(ce, re)
    return out


# speedup vs baseline: 1.7615x; 1.7615x over previous
import jax
import jax.numpy as jnp
from jax.experimental import pallas as pl
from jax.experimental.pallas import tpu as pltpu

_NPF = 256
_NPX = 26
_NPY = 230


def _pos_kernel(b, h, w, apad_ref, out_ref, pos, sems):
    hw = h * w
    r = jax.lax.broadcasted_iota(jnp.int32, (w + h, hw), 0)
    k = jax.lax.broadcasted_iota(jnp.int32, (w + h, hw), 1)
    ge = jnp.right_shift(r - w, 31) + 1
    v = (k % w) * (1 - ge) + (k // w) * ge
    t = r - w * ge
    b2 = (1 - jnp.minimum(jnp.abs(v - t), 1)).astype(jnp.float32)
    pos[...] = jax.lax.dot_general(
        apad_ref[...], b2, (((0,), (0,)), ((), ())),
        preferred_element_type=jnp.float32,
        precision=jax.lax.Precision.HIGHEST,
    )
    for i in range(b):
        pltpu.make_async_copy(pos, out_ref.at[i], sems.at[i]).start()
    for i in range(b):
        pltpu.make_async_copy(pos, out_ref.at[i], sems.at[i]).wait()


def kernel(x, row_embed, col_embed):
    b = x.shape[0]
    h, w = x.shape[-2], x.shape[-1]
    ce = col_embed[:w]
    re = row_embed[:h]
    top = jnp.pad(ce, ((0, 0), (0, _NPY)))
    bot = jnp.pad(re, ((0, 0), (_NPX, 0)))
    apad = jnp.concatenate([top, bot], axis=0)

    import functools
    out = pl.pallas_call(
        functools.partial(_pos_kernel, b, h, w),
        in_specs=[pl.BlockSpec(memory_space=pltpu.VMEM)],
        out_specs=pl.BlockSpec(memory_space=pl.ANY),
        out_shape=jax.ShapeDtypeStruct((b, _NPF, h * w), jnp.float32),
        scratch_shapes=[
            pltpu.VMEM((_NPF, h * w), jnp.float32),
            pltpu.SemaphoreType.DMA((b,)),
        ],
    )(apad)
    return out.reshape(b, _NPF, h, w)
